# Initial kernel scaffold; baseline (speedup 1.0000x reference)
import jax
import jax.numpy as jnp
from jax.experimental import pallas as pl


def kernel(x, edge_index, Wl0, bl0, Wr0, Wl1, bl1, Wr1, Wl2, bl2, Wr2):
    raise NotImplementedError("write your pallas kernel here")


# speedup vs baseline: 2.5161x; 2.5161x over previous
import jax
import jax.numpy as jnp
from jax import lax
from jax.experimental import pallas as pl
from jax.experimental.pallas import tpu as pltpu
from jax.experimental.pallas import tpu_sc as plsc

N = 10000
E = 320000
D = 128
H = 128

NC = 2
NS = 16
NHALF = N // NC
NPAD = NHALF + 8
CH = 128
CPT = 157
EPT = CPT * CH
EPAD = NS * EPT
RPT = 312
TAIL = NPAD - NS * RPT
ZROWS = 104
CW = 128


def _segsum_body(src_hbm, dstm_hbm, g_hbm, zero_hbm, out_hbm,
                 sidx, didx, rows, zbuf, acc, sem):
    c = lax.axis_index("c")
    s = lax.axis_index("s")
    ebase = s * EPT
    r0 = s * RPT

    pltpu.sync_copy(zero_hbm, zbuf)
    for t in range(RPT // ZROWS):
        pltpu.sync_copy(zbuf, acc.at[pl.ds(r0 + t * ZROWS, ZROWS)])

    @pl.when(s == 0)
    def _():
        pltpu.sync_copy(zbuf.at[pl.ds(0, TAIL)], acc.at[pl.ds(NS * RPT, TAIL)])

    plsc.subcore_barrier()

    @pl.loop(0, CPT)
    def step(j):
        base = ebase + j * CH
        pltpu.sync_copy(src_hbm.at[pl.ds(base, CH)], sidx)
        pltpu.sync_copy(dstm_hbm.at[c, pl.ds(base, CH)], didx.at[0])
        pltpu.async_copy(g_hbm.at[sidx], rows, sem).wait()
        pltpu.sync_copy(rows, acc.at[didx.at[0]], add=True)

    plsc.subcore_barrier()

    pltpu.sync_copy(acc.at[pl.ds(r0, RPT)], out_hbm.at[c, pl.ds(r0, RPT)])

    @pl.when(s == 0)
    def _():
        pltpu.sync_copy(acc.at[pl.ds(NS * RPT, TAIL)],
                        out_hbm.at[c, pl.ds(NS * RPT, TAIL)])


def _make_segsum():
    mesh = plsc.VectorSubcoreMesh(core_axis_name="c", subcore_axis_name="s")
    scratch = [
        pltpu.VMEM((CH,), jnp.int32),
        pltpu.VMEM((1, CH), jnp.int32),
        pltpu.VMEM((CH, H), jnp.float32),
        pltpu.VMEM((ZROWS, H), jnp.float32),
        pltpu.VMEM_SHARED((NPAD, H), jnp.float32),
        pltpu.SemaphoreType.DMA,
    ]
    return pl.kernel(
        _segsum_body,
        out_type=jax.ShapeDtypeStruct((NC, NPAD, H), jnp.float32),
        mesh=mesh,
        scratch_types=scratch,
    )


def _counts_body(dstm_hbm, zcnt_hbm, ones_hbm, cnt_hbm,
                 didx, ones, zcnt, cntacc):
    c = lax.axis_index("c")
    s = lax.axis_index("s")
    ebase = s * EPT
    r0 = s * RPT

    pltpu.sync_copy(zcnt_hbm, zcnt)
    pltpu.sync_copy(ones_hbm, ones)
    pltpu.sync_copy(zcnt, cntacc.at[pl.ds(r0, RPT)])

    @pl.when(s == 0)
    def _():
        pltpu.sync_copy(zcnt.at[pl.ds(0, TAIL)],
                        cntacc.at[pl.ds(NS * RPT, TAIL)])

    plsc.subcore_barrier()

    @pl.loop(0, CPT)
    def step(j):
        base = ebase + j * CH
        pltpu.sync_copy(dstm_hbm.at[c, pl.ds(base, CH)], didx.at[0])
        pltpu.sync_copy(ones, cntacc.at[didx.at[0]], add=True)

    plsc.subcore_barrier()

    pltpu.sync_copy(cntacc.at[pl.ds(r0, RPT)], cnt_hbm.at[c, pl.ds(r0, RPT)])

    @pl.when(s == 0)
    def _():
        pltpu.sync_copy(cntacc.at[pl.ds(NS * RPT, TAIL)],
                        cnt_hbm.at[c, pl.ds(NS * RPT, TAIL)])


def _make_counts():
    mesh = plsc.VectorSubcoreMesh(core_axis_name="c", subcore_axis_name="s")
    scratch = [
        pltpu.VMEM((1, CH), jnp.int32),
        pltpu.VMEM((CH, CW), jnp.float32),
        pltpu.VMEM((RPT, CW), jnp.float32),
        pltpu.VMEM_SHARED((NPAD, CW), jnp.float32),
    ]
    return pl.kernel(
        _counts_body,
        out_type=jax.ShapeDtypeStruct((NC, NPAD, CW), jnp.float32),
        mesh=mesh,
        scratch_types=scratch,
    )


_segsum = _make_segsum()
_counts = _make_counts()


BN = 1000
GRID = N // BN


def _mm_body(h_ref, w_ref, b_ref, g_ref, r_ref):
    res = (
        jnp.dot(h_ref[...], w_ref[...], preferred_element_type=jnp.float32)
        + b_ref[...]
    )
    g_ref[...] = res[:, :H]
    r_ref[...] = res[:, H:]


def _mm(h, w, b):
    return pl.pallas_call(
        _mm_body,
        grid=(GRID,),
        in_specs=[
            pl.BlockSpec((BN, H), lambda i: (i, 0)),
            pl.BlockSpec((H, 2 * H), lambda i: (0, 0)),
            pl.BlockSpec((1, 2 * H), lambda i: (0, 0)),
        ],
        out_specs=[
            pl.BlockSpec((BN, H), lambda i: (i, 0)),
            pl.BlockSpec((BN, H), lambda i: (i, 0)),
        ],
        out_shape=[
            jax.ShapeDtypeStruct((N, H), jnp.float32),
            jax.ShapeDtypeStruct((N, H), jnp.float32),
        ],
    )(h, w, b)


def _layer_body(acc_ref, cnt_ref, r_ref, w_ref, b_ref, g_ref, r2_ref):
    cnt = cnt_ref[:, 0:1]
    mean = acc_ref[...] / jnp.maximum(cnt, 1.0)
    h = jnp.maximum(mean + r_ref[...], 0.0)
    res = (
        jnp.dot(h, w_ref[...], preferred_element_type=jnp.float32) + b_ref[...]
    )
    g_ref[...] = res[:, :H]
    r2_ref[...] = res[:, H:]


def _layer(acc, cnt, r, w, b):
    return pl.pallas_call(
        _layer_body,
        grid=(GRID,),
        in_specs=[
            pl.BlockSpec((BN, H), lambda i: (i, 0)),
            pl.BlockSpec((BN, CW), lambda i: (i, 0)),
            pl.BlockSpec((BN, H), lambda i: (i, 0)),
            pl.BlockSpec((H, 2 * H), lambda i: (0, 0)),
            pl.BlockSpec((1, 2 * H), lambda i: (0, 0)),
        ],
        out_specs=[
            pl.BlockSpec((BN, H), lambda i: (i, 0)),
            pl.BlockSpec((BN, H), lambda i: (i, 0)),
        ],
        out_shape=[
            jax.ShapeDtypeStruct((N, H), jnp.float32),
            jax.ShapeDtypeStruct((N, H), jnp.float32),
        ],
    )(acc, cnt, r, w, b)


def _final_body(acc_ref, cnt_ref, r_ref, out_ref):
    @pl.when(pl.program_id(0) == 0)
    def _():
        out_ref[...] = jnp.zeros_like(out_ref)

    cnt = cnt_ref[:, 0:1]
    val = acc_ref[...] / jnp.maximum(cnt, 1.0) + r_ref[...]
    out_ref[...] += jnp.sum(val, axis=0, keepdims=True) * (1.0 / N)


def _final(acc, cnt, r):
    return pl.pallas_call(
        _final_body,
        grid=(GRID,),
        in_specs=[
            pl.BlockSpec((BN, H), lambda i: (i, 0)),
            pl.BlockSpec((BN, CW), lambda i: (i, 0)),
            pl.BlockSpec((BN, H), lambda i: (i, 0)),
        ],
        out_specs=pl.BlockSpec((1, H), lambda i: (0, 0)),
        out_shape=jax.ShapeDtypeStruct((1, H), jnp.float32),
    )(acc, cnt, r)


def _merge(parts):
    return jnp.concatenate([parts[0, :NHALF], parts[1, :NHALF]], axis=0)


@jax.jit
def kernel(x, edge_index, Wl0, bl0, Wr0, Wl1, bl1, Wr1, Wl2, bl2, Wr2):
    pad = EPAD - E
    src = jnp.concatenate([edge_index[0], jnp.zeros((pad,), jnp.int32)])
    dst = jnp.concatenate([edge_index[1], jnp.full((pad,), N, jnp.int32)])
    halves = []
    for c in range(NC):
        d = dst - c * NHALF
        halves.append(jnp.where((d < 0) | (d >= NHALF), NHALF, d))
    dstm = jnp.stack(halves)

    zero_seg = jnp.zeros((ZROWS, H), jnp.float32)
    zero_cnt = jnp.zeros((RPT, CW), jnp.float32)
    ones_rows = jnp.ones((CH, CW), jnp.float32)

    z = jnp.zeros((H,), jnp.float32)
    w0 = jnp.concatenate([Wl0.T, Wr0.T], axis=1)
    b0 = jnp.concatenate([z, bl0])[None, :]
    w1 = jnp.concatenate([Wl1.T, Wr1.T], axis=1)
    b1 = jnp.concatenate([z, bl1])[None, :]
    w2 = jnp.concatenate([Wl2.T, Wr2.T], axis=1)
    b2 = jnp.concatenate([z, bl2])[None, :]

    cnt = _merge(_counts(dstm, zero_cnt, ones_rows))
    g0, r0 = _mm(x, w0, b0)
    acc0 = _merge(_segsum(src, dstm, g0, zero_seg))
    g1, r1 = _layer(acc0, cnt, r0, w1, b1)
    acc1 = _merge(_segsum(src, dstm, g1, zero_seg))
    g2, r2 = _layer(acc1, cnt, r1, w2, b2)
    acc2 = _merge(_segsum(src, dstm, g2, zero_seg))
    return _final(acc2, cnt, r2)


# speedup vs baseline: 3.3295x; 1.3232x over previous
import jax
import jax.numpy as jnp
from jax import lax
from jax.experimental import pallas as pl
from jax.experimental.pallas import tpu as pltpu
from jax.experimental.pallas import tpu_sc as plsc

N = 10000
E = 320000
D = 128
H = 128

NC = 2
NS = 16
NHALF = N // NC
NPAD = NHALF + 8
CH = 128
CPT = 157
EPT = CPT * CH
EPAD = NS * EPT
RPT = 312
TAIL = NPAD - NS * RPT
ZROWS = 104
CW = 128


def _segsum_body(src_hbm, dstm_hbm, g_hbm, zero_hbm, out_hbm,
                 sidx, didx, rows, sidx2, didx2, rows2, zbuf, acc,
                 sem, sem2):
    c = lax.axis_index("c")
    s = lax.axis_index("s")
    ebase = s * EPT
    r0 = s * RPT

    pltpu.sync_copy(zero_hbm, zbuf)
    for t in range(RPT // ZROWS):
        pltpu.sync_copy(zbuf, acc.at[pl.ds(r0 + t * ZROWS, ZROWS)])

    @pl.when(s == 0)
    def _():
        pltpu.sync_copy(zbuf.at[pl.ds(0, TAIL)], acc.at[pl.ds(NS * RPT, TAIL)])

    plsc.subcore_barrier()

    def load(j, sid, did):
        base = ebase + j * CH
        pltpu.sync_copy(src_hbm.at[pl.ds(base, CH)], sid)
        pltpu.sync_copy(dstm_hbm.at[c, pl.ds(base, CH)], did.at[0])

    load(0, sidx, didx)
    pltpu.async_copy(g_hbm.at[sidx], rows, sem)

    @pl.loop(0, (CPT - 1) // 2)
    def step(p):
        j = 2 * p
        load(j + 1, sidx2, didx2)
        pltpu.async_copy(g_hbm.at[sidx2], rows2, sem2)
        pltpu.make_async_copy(g_hbm.at[sidx], rows, sem).wait()
        pltpu.sync_copy(rows, acc.at[didx.at[0]], add=True)
        load(j + 2, sidx, didx)
        pltpu.async_copy(g_hbm.at[sidx], rows, sem)
        pltpu.make_async_copy(g_hbm.at[sidx2], rows2, sem2).wait()
        pltpu.sync_copy(rows2, acc.at[didx2.at[0]], add=True)

    pltpu.make_async_copy(g_hbm.at[sidx], rows, sem).wait()
    pltpu.sync_copy(rows, acc.at[didx.at[0]], add=True)

    plsc.subcore_barrier()

    pltpu.sync_copy(acc.at[pl.ds(r0, RPT)], out_hbm.at[c, pl.ds(r0, RPT)])

    @pl.when(s == 0)
    def _():
        pltpu.sync_copy(acc.at[pl.ds(NS * RPT, TAIL)],
                        out_hbm.at[c, pl.ds(NS * RPT, TAIL)])


def _make_segsum():
    mesh = plsc.VectorSubcoreMesh(core_axis_name="c", subcore_axis_name="s")
    scratch = [
        pltpu.VMEM((CH,), jnp.int32),
        pltpu.VMEM((1, CH), jnp.int32),
        pltpu.VMEM((CH, H), jnp.float32),
        pltpu.VMEM((CH,), jnp.int32),
        pltpu.VMEM((1, CH), jnp.int32),
        pltpu.VMEM((CH, H), jnp.float32),
        pltpu.VMEM((ZROWS, H), jnp.float32),
        pltpu.VMEM_SHARED((NPAD, H), jnp.float32),
        pltpu.SemaphoreType.DMA,
        pltpu.SemaphoreType.DMA,
    ]
    return pl.kernel(
        _segsum_body,
        out_type=jax.ShapeDtypeStruct((NC, NPAD, H), jnp.float32),
        mesh=mesh,
        scratch_types=scratch,
    )


def _counts_body(dstm_hbm, zcnt_hbm, ones_hbm, cnt_hbm,
                 didx, ones, zcnt, cntacc):
    c = lax.axis_index("c")
    s = lax.axis_index("s")
    ebase = s * EPT
    r0 = s * RPT

    pltpu.sync_copy(zcnt_hbm, zcnt)
    pltpu.sync_copy(ones_hbm, ones)
    pltpu.sync_copy(zcnt, cntacc.at[pl.ds(r0, RPT)])

    @pl.when(s == 0)
    def _():
        pltpu.sync_copy(zcnt.at[pl.ds(0, TAIL)],
                        cntacc.at[pl.ds(NS * RPT, TAIL)])

    plsc.subcore_barrier()

    @pl.loop(0, CPT)
    def step(j):
        base = ebase + j * CH
        pltpu.sync_copy(dstm_hbm.at[c, pl.ds(base, CH)], didx.at[0])
        pltpu.sync_copy(ones, cntacc.at[didx.at[0]], add=True)

    plsc.subcore_barrier()

    pltpu.sync_copy(cntacc.at[pl.ds(r0, RPT)], cnt_hbm.at[c, pl.ds(r0, RPT)])

    @pl.when(s == 0)
    def _():
        pltpu.sync_copy(cntacc.at[pl.ds(NS * RPT, TAIL)],
                        cnt_hbm.at[c, pl.ds(NS * RPT, TAIL)])


def _make_counts():
    mesh = plsc.VectorSubcoreMesh(core_axis_name="c", subcore_axis_name="s")
    scratch = [
        pltpu.VMEM((1, CH), jnp.int32),
        pltpu.VMEM((CH, CW), jnp.float32),
        pltpu.VMEM((RPT, CW), jnp.float32),
        pltpu.VMEM_SHARED((NPAD, CW), jnp.float32),
    ]
    return pl.kernel(
        _counts_body,
        out_type=jax.ShapeDtypeStruct((NC, NPAD, CW), jnp.float32),
        mesh=mesh,
        scratch_types=scratch,
    )


_segsum = _make_segsum()
_counts = _make_counts()


BN = 1000
GRID = N // BN


def _mm_body(h_ref, w_ref, b_ref, g_ref, r_ref):
    res = (
        jnp.dot(h_ref[...], w_ref[...], preferred_element_type=jnp.float32)
        + b_ref[...]
    )
    g_ref[...] = res[:, :H]
    r_ref[...] = res[:, H:]


def _mm(h, w, b):
    return pl.pallas_call(
        _mm_body,
        grid=(GRID,),
        in_specs=[
            pl.BlockSpec((BN, H), lambda i: (i, 0)),
            pl.BlockSpec((H, 2 * H), lambda i: (0, 0)),
            pl.BlockSpec((1, 2 * H), lambda i: (0, 0)),
        ],
        out_specs=[
            pl.BlockSpec((BN, H), lambda i: (i, 0)),
            pl.BlockSpec((BN, H), lambda i: (i, 0)),
        ],
        out_shape=[
            jax.ShapeDtypeStruct((N, H), jnp.float32),
            jax.ShapeDtypeStruct((N, H), jnp.float32),
        ],
    )(h, w, b)


def _layer_body(acc_ref, cnt_ref, r_ref, w_ref, b_ref, g_ref, r2_ref):
    cnt = cnt_ref[:, 0:1]
    mean = acc_ref[...] / jnp.maximum(cnt, 1.0)
    h = jnp.maximum(mean + r_ref[...], 0.0)
    res = (
        jnp.dot(h, w_ref[...], preferred_element_type=jnp.float32) + b_ref[...]
    )
    g_ref[...] = res[:, :H]
    r2_ref[...] = res[:, H:]


def _layer(acc, cnt, r, w, b):
    return pl.pallas_call(
        _layer_body,
        grid=(GRID,),
        in_specs=[
            pl.BlockSpec((BN, H), lambda i: (i, 0)),
            pl.BlockSpec((BN, CW), lambda i: (i, 0)),
            pl.BlockSpec((BN, H), lambda i: (i, 0)),
            pl.BlockSpec((H, 2 * H), lambda i: (0, 0)),
            pl.BlockSpec((1, 2 * H), lambda i: (0, 0)),
        ],
        out_specs=[
            pl.BlockSpec((BN, H), lambda i: (i, 0)),
            pl.BlockSpec((BN, H), lambda i: (i, 0)),
        ],
        out_shape=[
            jax.ShapeDtypeStruct((N, H), jnp.float32),
            jax.ShapeDtypeStruct((N, H), jnp.float32),
        ],
    )(acc, cnt, r, w, b)


def _final_body(acc_ref, cnt_ref, r_ref, out_ref):
    @pl.when(pl.program_id(0) == 0)
    def _():
        out_ref[...] = jnp.zeros_like(out_ref)

    cnt = cnt_ref[:, 0:1]
    val = acc_ref[...] / jnp.maximum(cnt, 1.0) + r_ref[...]
    out_ref[...] += jnp.sum(val, axis=0, keepdims=True) * (1.0 / N)


def _final(acc, cnt, r):
    return pl.pallas_call(
        _final_body,
        grid=(GRID,),
        in_specs=[
            pl.BlockSpec((BN, H), lambda i: (i, 0)),
            pl.BlockSpec((BN, CW), lambda i: (i, 0)),
            pl.BlockSpec((BN, H), lambda i: (i, 0)),
        ],
        out_specs=pl.BlockSpec((1, H), lambda i: (0, 0)),
        out_shape=jax.ShapeDtypeStruct((1, H), jnp.float32),
    )(acc, cnt, r)


def _merge(parts):
    return jnp.concatenate([parts[0, :NHALF], parts[1, :NHALF]], axis=0)


@jax.jit
def kernel(x, edge_index, Wl0, bl0, Wr0, Wl1, bl1, Wr1, Wl2, bl2, Wr2):
    pad = EPAD - E
    src = jnp.concatenate([edge_index[0], jnp.zeros((pad,), jnp.int32)])
    dst = jnp.concatenate([edge_index[1], jnp.full((pad,), N, jnp.int32)])
    halves = []
    for c in range(NC):
        d = dst - c * NHALF
        halves.append(jnp.where((d < 0) | (d >= NHALF), NHALF, d))
    dstm = jnp.stack(halves)

    zero_seg = jnp.zeros((ZROWS, H), jnp.float32)
    zero_cnt = jnp.zeros((RPT, CW), jnp.float32)
    ones_rows = jnp.ones((CH, CW), jnp.float32)

    z = jnp.zeros((H,), jnp.float32)
    w0 = jnp.concatenate([Wl0.T, Wr0.T], axis=1)
    b0 = jnp.concatenate([z, bl0])[None, :]
    w1 = jnp.concatenate([Wl1.T, Wr1.T], axis=1)
    b1 = jnp.concatenate([z, bl1])[None, :]
    w2 = jnp.concatenate([Wl2.T, Wr2.T], axis=1)
    b2 = jnp.concatenate([z, bl2])[None, :]

    cnt = _merge(_counts(dstm, zero_cnt, ones_rows))
    g0, r0 = _mm(x, w0, b0)
    acc0 = _merge(_segsum(src, dstm, g0, zero_seg))
    g1, r1 = _layer(acc0, cnt, r0, w1, b1)
    acc1 = _merge(_segsum(src, dstm, g1, zero_seg))
    g2, r2 = _layer(acc1, cnt, r1, w2, b2)
    acc2 = _merge(_segsum(src, dstm, g2, zero_seg))
    return _final(acc2, cnt, r2)
